# Initial kernel scaffold; baseline (speedup 1.0000x reference)
import jax
import jax.numpy as jnp
from jax.experimental import pallas as pl


def kernel(feat, edge_index1, edge_index2, W1, al1, ar1, W2, al2, ar2):
    raise NotImplementedError("write your pallas kernel here")


# speedup vs baseline: 30.9677x; 30.9677x over previous
import functools

import jax
import jax.numpy as jnp
from jax import lax
from jax.experimental import pallas as pl
from jax.experimental.pallas import tpu as pltpu
from jax.experimental.pallas import tpu_sc as plsc

_N = 10000
_E = 320000
_D = 128
_H1 = 4
_NC = 2
_NS = 16
_NW = _NC * _NS
_CH = 80
_EPW = _E // _NW
_NCHUNK = _EPW // _CH
_NPAD = 10240
_ZR = 80

_SC_PARAMS = pltpu.CompilerParams(
    needs_layout_passes=False, use_tc_tiling_on_sc=False)


def _mesh():
    return plsc.VectorSubcoreMesh(
        core_axis_name="c", subcore_axis_name="s",
        num_cores=_NC, num_subcores=_NS)


def _make_denom_kernel(H):
    NH = _N * H
    CHH = _CH * H

    @functools.partial(
        pl.kernel,
        out_type=(
            jax.ShapeDtypeStruct((_NW, NH), jnp.float32),
            jax.ShapeDtypeStruct((_NW, _EPW * H), jnp.float32),
        ),
        mesh=_mesh(),
        compiler_params=_SC_PARAMS,
        scratch_types=[
            pltpu.VMEM((NH,), jnp.float32),
            pltpu.VMEM((NH,), jnp.float32),
            pltpu.VMEM((NH,), jnp.float32),
            pltpu.VMEM((CHH,), jnp.float32),
            pltpu.VMEM((1, _CH), jnp.int32),
            pltpu.VMEM((1, _CH), jnp.int32),
        ],
    )
    def dkern(el_hbm, er_hbm, src_hbm, dst_hbm, dsum_hbm, ex_hbm,
              el_v, er_v, dacc, exbuf, sbuf, dbuf):
        c = lax.axis_index("c")
        s = lax.axis_index("s")
        w = s * _NC + c

        pltpu.sync_copy(el_hbm, el_v)
        pltpu.sync_copy(er_hbm, er_v)

        def zbody(i, carry):
            dacc[pl.ds(i * 16, 16)] = jnp.zeros((16,), jnp.float32)
            return carry
        lax.fori_loop(0, NH // 16, zbody, 0)

        lanes = lax.iota(jnp.int32, 16)

        def chunk(j, carry):
            row = w * _NCHUNK + j
            pltpu.sync_copy(src_hbm.at[pl.ds(row, 1), :], sbuf)
            pltpu.sync_copy(dst_hbm.at[pl.ds(row, 1), :], dbuf)

            def qbody(q, qcarry):
                s16 = sbuf[0, pl.ds(q * 16, 16)]
                d16 = dbuf[0, pl.ds(q * 16, 16)]
                k16 = lanes + q * 16
                for h in range(H):
                    elv = plsc.load_gather(el_v, [s16 * H + h])
                    erv = plsc.load_gather(er_v, [d16 * H + h])
                    e = elv + erv
                    e = jnp.where(e >= 0.0, e, e * jnp.float32(0.2))
                    ex = jnp.exp(e)
                    plsc.addupdate_scatter(dacc, [d16 * H + h], ex)
                    if H == 1:
                        exbuf[pl.ds(q * 16, 16)] = ex
                    else:
                        plsc.store_scatter(exbuf, [k16 * H + h], ex)
                return qcarry
            lax.fori_loop(0, _CH // 16, qbody, 0)

            pltpu.sync_copy(exbuf, ex_hbm.at[w, pl.ds(j * CHH, CHH)])
            return carry
        lax.fori_loop(0, _NCHUNK, chunk, 0)

        pltpu.sync_copy(dacc, dsum_hbm.at[w])

    return dkern


def _make_agg_kernel(H):
    Dh = _D // H
    CHH = _CH * H

    @functools.partial(
        pl.kernel,
        out_type=jax.ShapeDtypeStruct((_NC, _NPAD, _D), jnp.float32),
        mesh=_mesh(),
        compiler_params=_SC_PARAMS,
        scratch_types=[
            pltpu.VMEM((CHH + 16,), jnp.float32),
            pltpu.VMEM((_CH, _D), jnp.float32),
            pltpu.VMEM((1, _CH), jnp.int32),
            pltpu.VMEM((1, _CH), jnp.int32),
            pltpu.VMEM((_ZR, _D), jnp.float32),
            pltpu.VMEM_SHARED((_NPAD, _D), jnp.float32),
            pltpu.SemaphoreType.DMA,
        ],
    )
    def bkern(proj_hbm, ex_hbm, src_hbm, dst_hbm, out_hbm,
              exg, rows, sbuf, dbuf, zbuf, acc_sh, sem):
        c = lax.axis_index("c")
        s = lax.axis_index("s")
        w = s * _NC + c

        def zrow(i, carry):
            zbuf[i // 8, pl.ds((i % 8) * 16, 16)] = jnp.zeros((16,), jnp.float32)
            return carry
        lax.fori_loop(0, _ZR * (_D // 16), zrow, 0)
        rpt = _NPAD // _NS
        for t in range(rpt // _ZR):
            pltpu.sync_copy(zbuf, acc_sh.at[pl.ds(s * rpt + t * _ZR, _ZR), :])
        plsc.subcore_barrier()

        def chunk(j, carry):
            row = w * _NCHUNK + j
            pltpu.sync_copy(src_hbm.at[pl.ds(row, 1), :], sbuf)
            pltpu.sync_copy(dst_hbm.at[pl.ds(row, 1), :], dbuf)
            pltpu.sync_copy(ex_hbm.at[w, pl.ds(j * CHH, CHH)],
                            exg.at[pl.ds(0, CHH)])

            pltpu.async_copy(proj_hbm.at[sbuf.at[0]], rows, sem).wait()

            def scale(b, scarry):
                for h in range(H):
                    a_s = exg[pl.ds(b * H + h, 16)][0]
                    for kk in range(Dh // 16):
                        off = h * Dh + kk * 16
                        rows[b, pl.ds(off, 16)] = rows[b, pl.ds(off, 16)] * a_s
                return scarry
            lax.fori_loop(0, _CH, scale, 0)

            pltpu.sync_copy(rows, acc_sh.at[dbuf.at[0]], add=True)
            return carry
        lax.fori_loop(0, _NCHUNK, chunk, 0)

        plsc.subcore_barrier()
        pltpu.sync_copy(acc_sh.at[pl.ds(s * rpt, rpt), :],
                        out_hbm.at[c, pl.ds(s * rpt, rpt), :])

    return bkern


_BN = 1000


def _tc1(feat, W1, alv, arv):
    def body(f_ref, w_ref, al_ref, ar_ref, p_ref, el_ref, er_ref):
        p = jnp.dot(f_ref[...], w_ref[...], preferred_element_type=jnp.float32)
        p_ref[...] = p
        el_ref[...] = jnp.dot(p, al_ref[...], preferred_element_type=jnp.float32)
        er_ref[...] = jnp.dot(p, ar_ref[...], preferred_element_type=jnp.float32)

    return pl.pallas_call(
        body,
        grid=(_N // _BN,),
        in_specs=[
            pl.BlockSpec((_BN, _D), lambda i: (i, 0)),
            pl.BlockSpec((_D, _D), lambda i: (0, 0)),
            pl.BlockSpec((_D, _H1), lambda i: (0, 0)),
            pl.BlockSpec((_D, _H1), lambda i: (0, 0)),
        ],
        out_specs=[
            pl.BlockSpec((_BN, _D), lambda i: (i, 0)),
            pl.BlockSpec((_BN, _H1), lambda i: (i, 0)),
            pl.BlockSpec((_BN, _H1), lambda i: (i, 0)),
        ],
        out_shape=[
            jax.ShapeDtypeStruct((_N, _D), jnp.float32),
            jax.ShapeDtypeStruct((_N, _H1), jnp.float32),
            jax.ShapeDtypeStruct((_N, _H1), jnp.float32),
        ],
    )(feat, W1, alv, arv)


def _tc2(acc1p, dsum1p, S1, W2, al2v, ar2v):
    def body(a_ref, d_ref, s_ref, w_ref, al_ref, ar_ref,
             p_ref, el_ref, er_ref):
        dsum = jnp.sum(d_ref[...], axis=0)
        rep = jnp.dot(dsum, s_ref[...],
                      preferred_element_type=jnp.float32)
        acc = a_ref[0] + a_ref[1]
        h1 = jnp.maximum(acc / (rep + jnp.float32(1e-9)), 0.0)
        p2 = jnp.dot(h1, w_ref[...], preferred_element_type=jnp.float32)
        p_ref[...] = p2
        el_ref[...] = jnp.dot(p2, al_ref[...], preferred_element_type=jnp.float32)
        er_ref[...] = jnp.dot(p2, ar_ref[...], preferred_element_type=jnp.float32)

    return pl.pallas_call(
        body,
        grid=(_N // _BN,),
        in_specs=[
            pl.BlockSpec((_NC, _BN, _D), lambda i: (0, i, 0)),
            pl.BlockSpec((_NW, _BN, _H1), lambda i: (0, i, 0)),
            pl.BlockSpec((_H1, _D), lambda i: (0, 0)),
            pl.BlockSpec((_D, _D), lambda i: (0, 0)),
            pl.BlockSpec((_D, 1), lambda i: (0, 0)),
            pl.BlockSpec((_D, 1), lambda i: (0, 0)),
        ],
        out_specs=[
            pl.BlockSpec((_BN, _D), lambda i: (i, 0)),
            pl.BlockSpec((_BN, 1), lambda i: (i, 0)),
            pl.BlockSpec((_BN, 1), lambda i: (i, 0)),
        ],
        out_shape=[
            jax.ShapeDtypeStruct((_N, _D), jnp.float32),
            jax.ShapeDtypeStruct((_N, 1), jnp.float32),
            jax.ShapeDtypeStruct((_N, 1), jnp.float32),
        ],
    )(acc1p, dsum1p, S1, W2, al2v, ar2v)


def _tc3(acc2p, dsum2p):
    def body(a_ref, d_ref, out_ref):
        dsum = jnp.sum(d_ref[...], axis=0)
        acc = a_ref[0] + a_ref[1]
        out_ref[...] = acc / (dsum + jnp.float32(1e-9))

    return pl.pallas_call(
        body,
        grid=(_N // _BN,),
        in_specs=[
            pl.BlockSpec((_NC, _BN, _D), lambda i: (0, i, 0)),
            pl.BlockSpec((_NW, _BN, 1), lambda i: (0, i, 0)),
        ],
        out_specs=pl.BlockSpec((_BN, _D), lambda i: (i, 0)),
        out_shape=jax.ShapeDtypeStruct((_N, _D), jnp.float32),
    )(acc2p, dsum2p)


_denom1 = _make_denom_kernel(_H1)
_agg1 = _make_agg_kernel(_H1)
_denom2 = _make_denom_kernel(1)
_agg2 = _make_agg_kernel(1)


def kernel(feat, edge_index1, edge_index2, W1, al1, ar1, W2, al2, ar2):
    src1 = edge_index1[0].reshape(_E // _CH, _CH)
    dst1 = edge_index1[1].reshape(_E // _CH, _CH)
    src2 = edge_index2[0].reshape(_E // _CH, _CH)
    dst2 = edge_index2[1].reshape(_E // _CH, _CH)

    eye = jnp.eye(_H1, dtype=jnp.float32)
    alv1 = (al1[:, :, None] * eye[:, None, :]).reshape(_D, _H1)
    arv1 = (ar1[:, :, None] * eye[:, None, :]).reshape(_D, _H1)
    S1 = jnp.kron(eye, jnp.ones((1, _D // _H1), dtype=jnp.float32))
    al2v = al2.reshape(_D, 1)
    ar2v = ar2.reshape(_D, 1)

    proj1, el1, er1 = _tc1(feat, W1, alv1, arv1)
    el1f = el1.reshape(_N * _H1)
    er1f = er1.reshape(_N * _H1)
    dsum1p, ex1 = _denom1(el1f, er1f, src1, dst1)
    acc1p = _agg1(proj1, ex1, src1, dst1)

    proj2, el2, er2 = _tc2(acc1p, dsum1p.reshape(_NW, _N, _H1),
                           S1, W2, al2v, ar2v)
    el2f = el2.reshape(_N)
    er2f = er2.reshape(_N)
    dsum2p, ex2 = _denom2(el2f, er2f, src2, dst2)
    acc2p = _agg2(proj2, ex2, src2, dst2)

    return _tc3(acc2p, dsum2p.reshape(_NW, _N, 1))
